# baseline (speedup 1.0000x reference)
import numpy as np
import jax
import jax.numpy as jnp
from jax import lax
from jax.experimental import pallas as pl
from jax.experimental.pallas import tpu as pltpu

NUM_ROWS = 16384
NUM_COLS = 1000
VIEW_ROWS = 16000
VIEW_COLS = 1024
BLOCK_VROWS = 1000
NUM_SLOTS = 4
NUM_CHUNKS = VIEW_ROWS // BLOCK_VROWS
NUM_ROUNDS = NUM_CHUNKS // NUM_SLOTS

_G = np.arange(VIEW_ROWS, dtype=np.int64)
_R0 = (_G * VIEW_COLS) // NUM_COLS
_R1 = np.minimum(_R0 + 1, NUM_ROWS - 1)
_R2 = np.minimum(_R0 + 2, NUM_ROWS - 1)
_CA = (_R0 * NUM_COLS - _G * VIEW_COLS).astype(np.int32)
_CB = (_R1 * NUM_COLS - _G * VIEW_COLS).astype(np.int32)
_CC = (_R2 * NUM_COLS - _G * VIEW_COLS).astype(np.int32)
_R0 = _R0.astype(np.int32)
_R1 = _R1.astype(np.int32)
_R2 = _R2.astype(np.int32)


def _copy(o_ref, buf_ref, sem_ref, k, ci):
    return pltpu.make_async_copy(
        buf_ref.at[k],
        o_ref.at[pl.ds(ci * BLOCK_VROWS, BLOCK_VROWS), :],
        sem_ref.at[k],
    )


def _onehot_body(a_ref, b_ref, c_ref, o_ref, buf_ref, sem_ref):
    def one_round(r, carry):
        for k in range(NUM_SLOTS):
            ci = r * NUM_SLOTS + k

            @pl.when(r > 0)
            def _wait_prev():
                _copy(o_ref, buf_ref, sem_ref, k, ci).wait()

            av = a_ref[ci, 0, :]
            bv = b_ref[ci, 0, :]
            cv = c_ref[ci, 0, :]
            cols = lax.broadcasted_iota(jnp.int32, (BLOCK_VROWS, VIEW_COLS), 1)
            eq = (
                (cols == av[:, None])
                | (cols == bv[:, None])
                | (cols == cv[:, None])
            )
            buf_ref[k] = eq.astype(jnp.float32)
            _copy(o_ref, buf_ref, sem_ref, k, ci).start()
        return carry

    lax.fori_loop(0, NUM_ROUNDS, one_round, 0)
    for k in range(NUM_SLOTS):
        ci = (NUM_ROUNDS - 1) * NUM_SLOTS + k
        _copy(o_ref, buf_ref, sem_ref, k, ci).wait()


def kernel(x):
    xi = x.astype(jnp.int32)
    a = (jnp.take(xi, _R0) + _CA).reshape(NUM_CHUNKS, 1, BLOCK_VROWS)
    b = (jnp.take(xi, _R1) + _CB).reshape(NUM_CHUNKS, 1, BLOCK_VROWS)
    c = (jnp.take(xi, _R2) + _CC).reshape(NUM_CHUNKS, 1, BLOCK_VROWS)
    out = pl.pallas_call(
        _onehot_body,
        in_specs=[
            pl.BlockSpec(memory_space=pltpu.VMEM),
            pl.BlockSpec(memory_space=pltpu.VMEM),
            pl.BlockSpec(memory_space=pltpu.VMEM),
        ],
        out_specs=pl.BlockSpec(memory_space=pl.ANY),
        out_shape=jax.ShapeDtypeStruct((VIEW_ROWS, VIEW_COLS), jnp.float32),
        scratch_shapes=[
            pltpu.VMEM((NUM_SLOTS, BLOCK_VROWS, VIEW_COLS), jnp.float32),
            pltpu.SemaphoreType.DMA((NUM_SLOTS,)),
        ],
    )(a, b, c)
    return out.reshape(NUM_ROWS, NUM_COLS)


# speedup vs baseline: 1.0772x; 1.0772x over previous
import numpy as np
import jax
import jax.numpy as jnp
from jax import lax
from jax.experimental import pallas as pl
from jax.experimental.pallas import tpu as pltpu

NUM_ROWS = 16384
NUM_COLS = 1000
VIEW_ROWS = 16000
VIEW_COLS = 1024
BLOCK_VROWS = 1000
BLOCK_LROWS = 1024
NUM_SLOTS = 4
NUM_CHUNKS = VIEW_ROWS // BLOCK_VROWS
NUM_ROUNDS = NUM_CHUNKS // NUM_SLOTS

_G = np.arange(VIEW_ROWS, dtype=np.int64)
_R0 = (_G * VIEW_COLS) // NUM_COLS
_CA = (_R0 * NUM_COLS - _G * VIEW_COLS).astype(np.int32).reshape(NUM_CHUNKS, BLOCK_VROWS)
_CB = _CA + NUM_COLS
_CC = _CB + NUM_COLS
_S = (_R0[:BLOCK_VROWS] - np.arange(BLOCK_VROWS, dtype=np.int64)).astype(np.int32)
_MAX_SHIFT = int(_S.max())


def _copy(o_ref, buf_ref, sem_ref, k, ci):
    return pltpu.make_async_copy(
        buf_ref.at[k],
        o_ref.at[pl.ds(ci * BLOCK_VROWS, BLOCK_VROWS), :],
        sem_ref.at[k],
    )


def _onehot_body(a_ref, b_ref, c_ref, o_ref, buf_ref, sem_ref):
    def one_round(r, carry):
        for k in range(NUM_SLOTS):
            ci = r * NUM_SLOTS + k

            @pl.when(r > 0)
            def _wait_prev():
                _copy(o_ref, buf_ref, sem_ref, k, ci).wait()

            av = a_ref[ci, 0, :]
            bv = b_ref[ci, 0, :]
            cv = c_ref[ci, 0, :]
            cols = lax.broadcasted_iota(jnp.int32, (BLOCK_VROWS, VIEW_COLS), 1)
            eq = (
                (cols == av[:, None])
                | (cols == bv[:, None])
                | (cols == cv[:, None])
            )
            buf_ref[k] = eq.astype(jnp.float32)
            _copy(o_ref, buf_ref, sem_ref, k, ci).start()
        return carry

    lax.fori_loop(0, NUM_ROUNDS, one_round, 0)
    for k in range(NUM_SLOTS):
        ci = (NUM_ROUNDS - 1) * NUM_SLOTS + k
        _copy(o_ref, buf_ref, sem_ref, k, ci).wait()


def _staircase(xc):
    outs = []
    for extra in range(3):
        acc = None
        for sh in range(_MAX_SHIFT + 1):
            y = lax.slice(xc, (sh + extra,), (sh + extra + NUM_ROWS,))
            y = y.reshape(NUM_CHUNKS, BLOCK_LROWS)[:, :BLOCK_VROWS]
            m = jnp.asarray(_S == sh)
            acc = jnp.where(m, y, acc) if acc is not None else y
        outs.append(acc)
    return outs


def kernel(x):
    xi = x.astype(jnp.int32)
    xc = jnp.concatenate([xi, jnp.zeros((BLOCK_LROWS,), jnp.int32)])
    g0, g1, g2 = _staircase(xc)
    a = (g0 + _CA).reshape(NUM_CHUNKS, 1, BLOCK_VROWS)
    b = (g1 + _CB).reshape(NUM_CHUNKS, 1, BLOCK_VROWS)
    c = (g2 + _CC).reshape(NUM_CHUNKS, 1, BLOCK_VROWS)
    out = pl.pallas_call(
        _onehot_body,
        in_specs=[
            pl.BlockSpec(memory_space=pltpu.VMEM),
            pl.BlockSpec(memory_space=pltpu.VMEM),
            pl.BlockSpec(memory_space=pltpu.VMEM),
        ],
        out_specs=pl.BlockSpec(memory_space=pl.ANY),
        out_shape=jax.ShapeDtypeStruct((VIEW_ROWS, VIEW_COLS), jnp.float32),
        scratch_shapes=[
            pltpu.VMEM((NUM_SLOTS, BLOCK_VROWS, VIEW_COLS), jnp.float32),
            pltpu.SemaphoreType.DMA((NUM_SLOTS,)),
        ],
    )(a, b, c)
    return out.reshape(NUM_ROWS, NUM_COLS)


# speedup vs baseline: 2.1965x; 2.0392x over previous
import jax
import jax.numpy as jnp
from jax import lax
from jax.experimental import pallas as pl
from jax.experimental.pallas import tpu as pltpu

NUM_ROWS = 16384
NUM_COLS = 1000
FAT_COLS = 896
TAIL_COLS = 128
WRAP_COLS = TAIL_COLS - (NUM_COLS - FAT_COLS)
BLOCK_ROWS = 512
NUM_SLOTS = 8
NUM_CHUNKS = NUM_ROWS // BLOCK_ROWS
NUM_ROUNDS = NUM_CHUNKS // NUM_SLOTS
LAST_CHUNK = NUM_CHUNKS - 1


def _chunk_copies(o_ref, fat_ref, tail_ref, sem_ref, k, ci, tail_rows, t_off):
    rows = pl.ds(ci * BLOCK_ROWS, BLOCK_ROWS)
    fat = pltpu.make_async_copy(
        fat_ref.at[k],
        o_ref.at[rows, pl.ds(0, FAT_COLS)],
        sem_ref.at[k, 0],
    )
    tail = pltpu.make_async_copy(
        tail_ref.at[k, pl.ds(0, tail_rows)],
        o_ref.at[pl.ds(ci * BLOCK_ROWS, tail_rows), pl.ds(t_off, TAIL_COLS)],
        sem_ref.at[k, 1],
    )
    return fat, tail


def _onehot_body(x_ref, xn_ref, o_ref, fat_ref, tail_ref, last_ref, sem_ref):
    def do_chunk(k, ci, tail_rows, t_off):
        xs = x_ref[0, pl.ds(ci * BLOCK_ROWS, BLOCK_ROWS)]
        xn = xn_ref[0, pl.ds(ci * BLOCK_ROWS, BLOCK_ROWS)]
        cols_f = lax.broadcasted_iota(jnp.int32, (BLOCK_ROWS, FAT_COLS), 1)
        fat_ref[k] = (cols_f == xs[:, None]).astype(jnp.float32)
        cols_t = lax.broadcasted_iota(jnp.int32, (BLOCK_ROWS, TAIL_COLS), 1)
        own = (cols_t + FAT_COLS) == xs[:, None]
        nxt = (cols_t - (TAIL_COLS - WRAP_COLS)) == xn[:, None]
        tail_ref[k] = (own | nxt).astype(jnp.float32)
        fat, tail = _chunk_copies(o_ref, fat_ref, tail_ref, sem_ref, k, ci, tail_rows, t_off)
        fat.start()
        tail.start()

    def one_round(r, carry):
        t_off = pl.multiple_of(carry, 128)
        for k in range(NUM_SLOTS):
            ci = r * NUM_SLOTS + k

            @pl.when(r > 0)
            def _wait_prev():
                fat, tail = _chunk_copies(
                    o_ref, fat_ref, tail_ref, sem_ref, k, ci, BLOCK_ROWS, t_off
                )
                fat.wait()
                tail.wait()

            do_chunk(k, ci, BLOCK_ROWS, t_off)
        return carry

    t_off = pl.multiple_of(
        lax.fori_loop(0, NUM_ROUNDS - 1, one_round, jnp.int32(FAT_COLS)), 128
    )
    r = NUM_ROUNDS - 1
    for k in range(NUM_SLOTS):
        ci = r * NUM_SLOTS + k
        fat, tail = _chunk_copies(
            o_ref, fat_ref, tail_ref, sem_ref, k, ci, BLOCK_ROWS, t_off
        )
        fat.wait()
        tail.wait()
        do_chunk(k, ci, BLOCK_ROWS if ci != LAST_CHUNK else BLOCK_ROWS - 8, t_off)

    xl = x_ref[0, pl.ds(NUM_ROWS - 8, 8)]
    cols_l = lax.broadcasted_iota(jnp.int32, (8, NUM_COLS - FAT_COLS), 1) + FAT_COLS
    last_ref[...] = (cols_l == xl[:, None]).astype(jnp.float32)
    last_copy = pltpu.make_async_copy(
        last_ref,
        o_ref.at[pl.ds(NUM_ROWS - 8, 8), pl.ds(FAT_COLS, NUM_COLS - FAT_COLS)],
        sem_ref.at[0, 0],
    )
    last_copy.start()
    last_copy.wait()

    for k in range(NUM_SLOTS):
        ci = (NUM_ROUNDS - 1) * NUM_SLOTS + k
        fat, tail = _chunk_copies(
            o_ref, fat_ref, tail_ref, sem_ref, k, ci,
            BLOCK_ROWS if ci != LAST_CHUNK else BLOCK_ROWS - 8, t_off,
        )
        fat.wait()
        tail.wait()


def kernel(x):
    xi = x.astype(jnp.int32)
    x2 = xi.reshape(1, NUM_ROWS)
    xn = jnp.concatenate([xi[1:], jnp.full((1,), -1, jnp.int32)]).reshape(1, NUM_ROWS)
    out = pl.pallas_call(
        _onehot_body,
        in_specs=[
            pl.BlockSpec(memory_space=pltpu.VMEM),
            pl.BlockSpec(memory_space=pltpu.VMEM),
        ],
        out_specs=pl.BlockSpec(memory_space=pl.ANY),
        out_shape=jax.ShapeDtypeStruct((NUM_ROWS, NUM_COLS), jnp.float32),
        scratch_shapes=[
            pltpu.VMEM((NUM_SLOTS, BLOCK_ROWS, FAT_COLS), jnp.float32),
            pltpu.VMEM((NUM_SLOTS, BLOCK_ROWS, TAIL_COLS), jnp.float32),
            pltpu.VMEM((8, NUM_COLS - FAT_COLS), jnp.float32),
            pltpu.SemaphoreType.DMA((NUM_SLOTS, 2)),
        ],
        compiler_params=pltpu.CompilerParams(disable_bounds_checks=True),
    )(x2, xn)
    return out


# speedup vs baseline: 2.2704x; 1.0336x over previous
import jax
import jax.numpy as jnp
from jax import lax
from jax.experimental import pallas as pl

NUM_ROWS = 16384
NUM_COLS = 1000
BLOCK_ROWS = 1024


def _onehot_body(x_ref, o_ref):
    i = pl.program_id(0)
    xs = x_ref[0, pl.ds(i * BLOCK_ROWS, BLOCK_ROWS)]
    cols = lax.broadcasted_iota(jnp.int32, (BLOCK_ROWS, NUM_COLS), 1)
    o_ref[...] = (cols == xs[:, None]).astype(jnp.float32)


def kernel(x):
    x2 = x.reshape(1, NUM_ROWS).astype(jnp.int32)
    out = pl.pallas_call(
        _onehot_body,
        grid=(NUM_ROWS // BLOCK_ROWS,),
        in_specs=[pl.BlockSpec((1, NUM_ROWS), lambda i: (0, 0))],
        out_specs=pl.BlockSpec((BLOCK_ROWS, NUM_COLS), lambda i: (i, 0)),
        out_shape=jax.ShapeDtypeStruct((NUM_ROWS, NUM_COLS), jnp.float32),
    )(x2)
    return out
